# baseline (speedup 1.0000x reference)
import functools

import jax
import jax.numpy as jnp
from jax import lax
from jax.experimental import pallas as pl
from jax.experimental.pallas import tpu as pltpu
from jax.experimental.pallas import tpu_sc as plsc

_B, _P, _D = 4096, 200, 64
_MAXLEN = 1000
_N = _B * _P
_NW = 32
_R = _N // _NW
_C = 512
_NCHUNK = _R // _C
_IDXW = 128
_KG = _C // _IDXW


def _sc_gather_add(pe2, idx2, x2):
    mesh = plsc.VectorSubcoreMesh(core_axis_name="c", subcore_axis_name="s")

    @functools.partial(
        pl.kernel,
        mesh=mesh,
        compiler_params=pltpu.CompilerParams(use_tc_tiling_on_sc=False),
        out_type=jax.ShapeDtypeStruct((_N, _D), jnp.float32),
        scratch_types=[
            pltpu.VMEM((_KG, _IDXW), jnp.int32),
            pltpu.VMEM((_C, _D), jnp.float32),
            pltpu.VMEM((_C, _D), jnp.float32),
            pltpu.SemaphoreType.DMA,
            pltpu.SemaphoreType.DMA,
        ],
    )
    def body(pe_hbm, idx_hbm, x_hbm, out_hbm, idx_v, rows_v, x_v, sem_g, sem_x):
        wid = lax.axis_index("s") * 2 + lax.axis_index("c")
        wbase = wid * _R

        def chunk_body(ci, carry):
            base = wbase + ci * _C
            for k in range(_KG):
                pltpu.sync_copy(
                    idx_hbm.at[pl.ds(base + k * _IDXW, _IDXW)], idx_v.at[k]
                )
            cx = pltpu.async_copy(x_hbm.at[pl.ds(base, _C)], x_v, sem_x)
            gathers = [
                pltpu.async_copy(
                    pe_hbm.at[idx_v.at[k]],
                    rows_v.at[pl.ds(k * _IDXW, _IDXW)],
                    sem_g,
                )
                for k in range(_KG)
            ]
            for g in gathers:
                g.wait()
            cx.wait()

            @plsc.parallel_loop(0, _C, unroll=8)
            def row_add(i):
                for j in range(_D // 16):
                    sl = pl.ds(j * 16, 16)
                    plsc.addupdate(x_v.at[i, sl], rows_v[i, sl])
            pltpu.sync_copy(x_v, out_hbm.at[pl.ds(base, _C)])
            return carry

        lax.fori_loop(0, _NCHUNK, chunk_body, 0)

    return body(pe2, idx2, x2)


def kernel(x, indices, pe):
    x2 = x.reshape(_N, _D)
    idx2 = indices.reshape(_N).astype(jnp.int32)
    pe2 = pe.reshape(_MAXLEN, _D)
    out = _sc_gather_add(pe2, idx2, x2)
    return out.reshape(_B, _P, _D)


# speedup vs baseline: 1.3684x; 1.3684x over previous
import functools

import jax
import jax.numpy as jnp
from jax import lax
from jax.experimental import pallas as pl
from jax.experimental.pallas import tpu as pltpu
from jax.experimental.pallas import tpu_sc as plsc

_B, _P, _D = 4096, 200, 64
_MAXLEN = 1000
_N = _B * _P
_NW = 32
_R = _N // _NW
_C = 256
_NCHUNK = _R // _C
_IDXW = 128
_KG = _C // _IDXW


def _sc_gather_add(pe2, idx2, x2):
    mesh = plsc.VectorSubcoreMesh(core_axis_name="c", subcore_axis_name="s")

    @functools.partial(
        pl.kernel,
        mesh=mesh,
        out_type=jax.ShapeDtypeStruct((_N, _D), jnp.float32),
        scratch_types=[
            pltpu.VMEM((_KG, _IDXW), jnp.int32),
            pltpu.VMEM((_C, 128), jnp.float32),
            pltpu.VMEM((_C, _D), jnp.float32),
            pltpu.SemaphoreType.DMA,
            pltpu.SemaphoreType.DMA,
        ],
    )
    def body(pe_hbm, idx_hbm, x_hbm, out_hbm, idx_v, rows_v, x_v, sem_g, sem_x):
        wid = lax.axis_index("s") * 2 + lax.axis_index("c")
        wbase = wid * _R

        def chunk_body(ci, carry):
            base = wbase + ci * _C
            for k in range(_KG):
                pltpu.sync_copy(
                    idx_hbm.at[pl.ds(base + k * _IDXW, _IDXW)], idx_v.at[k]
                )
            cx = pltpu.async_copy(x_hbm.at[pl.ds(base, _C)], x_v, sem_x)
            gathers = [
                pltpu.async_copy(
                    pe_hbm.at[idx_v.at[k]],
                    rows_v.at[pl.ds(k * _IDXW, _IDXW)],
                    sem_g,
                )
                for k in range(_KG)
            ]
            for g in gathers:
                g.wait()
            cx.wait()

            @plsc.parallel_loop(0, _C, unroll=8)
            def row_add(i):
                for j in range(_D // 16):
                    sl = pl.ds(j * 16, 16)
                    plsc.addupdate(x_v.at[i, sl], rows_v[i, sl])
            pltpu.sync_copy(x_v, out_hbm.at[pl.ds(base, _C)])
            return carry

        lax.fori_loop(0, _NCHUNK, chunk_body, 0)

    return body(pe2, idx2, x2)


def kernel(x, indices, pe):
    x2 = x.reshape(_N, _D)
    idx2 = indices.reshape(_N).astype(jnp.int32)
    pe2 = jnp.pad(pe.reshape(_MAXLEN, _D), ((0, 0), (0, 128 - _D)))
    out = _sc_gather_add(pe2, idx2, x2)
    return out.reshape(_B, _P, _D)
